# baseline (speedup 1.0000x reference)
import functools

import jax
import jax.numpy as jnp
from jax import lax
from jax.experimental import pallas as pl
from jax.experimental.pallas import tpu as pltpu
from jax.experimental.pallas import tpu_sc as plsc

B = 1024
S = 51
SP = 56
D = 128
NW = 32
BPW = B // NW
MPAD = 64
BPS = 2
SLOTS = BPW // BPS
NBUF = 4
L = 16


def _build():
    info = plsc.get_sparse_core_info()
    nc = info.num_cores
    mesh = plsc.VectorSubcoreMesh(core_axis_name="c", subcore_axis_name="s")

    @functools.partial(
        pl.kernel,
        mesh=mesh,
        out_type=jax.ShapeDtypeStruct((B, S, D), jnp.float32),
        scratch_types=[
            pltpu.VMEM((SLOTS, BPS * SP), jnp.int32),
            pltpu.VMEM((BPW * MPAD,), jnp.int32),
            pltpu.VMEM((NBUF, BPS * SP, D), jnp.float32),
        ]
        + [pltpu.SemaphoreType.DMA] * (2 * NBUF),
    )
    def k(table_hbm, idx_hbm, msk_hbm, out_hbm, idx_v, msk_v, rows_v, *sems):
        gsem = sems[:NBUF]
        ssem = sems[NBUF:]
        wid = lax.axis_index("s") * nc + lax.axis_index("c")
        pltpu.sync_copy(idx_hbm.at[wid], idx_v)
        for b in range(NBUF):
            pltpu.async_copy(table_hbm.at[idx_v.at[b]], rows_v.at[b], gsem[b])
        pltpu.sync_copy(msk_hbm.at[pl.ds(wid * BPW * MPAD, BPW * MPAD)], msk_v)
        zeros = jnp.zeros((L,), jnp.float32)

        def store_slot(buf, sl, sem):
            for h in range(BPS):
                pltpu.async_copy(
                    rows_v.at[buf, pl.ds(h * SP, S)],
                    out_hbm.at[wid * BPW + sl * BPS + h],
                    sem,
                )

        def wait_store_slot(buf, sl, sem):
            for h in range(BPS):
                pltpu.make_async_copy(
                    rows_v.at[buf, pl.ds(h * SP, S)],
                    out_hbm.at[wid * BPW + sl * BPS + h],
                    sem,
                ).wait()

        def body(p, carry):
            for b in range(NBUF):
                sl = p * NBUF + b
                prev = (b - 1) % NBUF

                @pl.when(jnp.logical_and(sl >= 1, sl - 1 + NBUF < SLOTS))
                def _(b=b, sl=sl, prev=prev):
                    with jax.named_scope("refill"):
                        wait_store_slot(prev, sl - 1, ssem[prev])
                        pltpu.async_copy(
                            table_hbm.at[idx_v.at[sl - 1 + NBUF]],
                            rows_v.at[prev],
                            gsem[prev],
                        )

                with jax.named_scope("gwait"):
                    pltpu.make_async_copy(
                        table_hbm.at[idx_v.at[sl]],
                        rows_v.at[b],
                        gsem[b],
                    ).wait()

                with jax.named_scope("maskz"):
                    for h in range(BPS):
                        for g in range(4):
                            nt = L if g < 3 else S - 3 * L
                            mv = msk_v[
                                pl.ds((sl * BPS + h) * MPAD + g * L, L)
                            ]
                            for t in range(nt):
                                r = h * SP + g * L + t

                                @pl.when(mv[t] != 0)
                                def _(r=r, b=b):
                                    for j in range(D // L):
                                        rows_v[b, r, pl.ds(j * L, L)] = zeros

                with jax.named_scope("sstart"):
                    store_slot(b, sl, ssem[b])

            return carry

        lax.fori_loop(0, SLOTS // NBUF, body, 0)
        for b in range(NBUF):
            wait_store_slot(b, SLOTS - NBUF + b, ssem[b])

    return k


_k = jax.jit(_build())


def kernel(node_ids, mask, emb_table):
    fill = (jnp.arange(B, dtype=jnp.int32)[:, None] * (SP - S)
            + jnp.arange(SP - S, dtype=jnp.int32)[None, :]) % (B * (SP - S))
    idx = jnp.concatenate([node_ids.astype(jnp.int32), fill], axis=1).reshape(
        NW, SLOTS, BPS * SP)
    msk = jnp.pad(mask.astype(jnp.int32), ((0, 0), (0, MPAD - S))).reshape(-1)
    return _k(emb_table, idx, msk)


# speedup vs baseline: 1.1014x; 1.1014x over previous
import functools

import jax
import jax.numpy as jnp
from jax import lax
from jax.experimental import pallas as pl
from jax.experimental.pallas import tpu as pltpu
from jax.experimental.pallas import tpu_sc as plsc

B = 1024
S = 51
SP = 56
D = 128
NW = 32
BPW = B // NW
MPAD = 64
BPS = 1
SLOTS = BPW // BPS
NBUF = 4
L = 16


def _build():
    info = plsc.get_sparse_core_info()
    nc = info.num_cores
    mesh = plsc.VectorSubcoreMesh(core_axis_name="c", subcore_axis_name="s")

    @functools.partial(
        pl.kernel,
        mesh=mesh,
        out_type=jax.ShapeDtypeStruct((B, S, D), jnp.float32),
        scratch_types=[
            pltpu.VMEM((SLOTS, BPS * SP), jnp.int32),
            pltpu.VMEM((BPW * MPAD,), jnp.int32),
            pltpu.VMEM((NBUF, BPS * SP, D), jnp.float32),
        ]
        + [pltpu.SemaphoreType.DMA] * (2 * NBUF),
    )
    def k(table_hbm, idx_hbm, msk_hbm, out_hbm, idx_v, msk_v, rows_v, *sems):
        gsem = sems[:NBUF]
        ssem = sems[NBUF:]
        wid = lax.axis_index("s") * nc + lax.axis_index("c")
        pltpu.sync_copy(idx_hbm.at[wid], idx_v)
        for b in range(NBUF):
            pltpu.async_copy(table_hbm.at[idx_v.at[b]], rows_v.at[b], gsem[b])
        pltpu.sync_copy(msk_hbm.at[pl.ds(wid * BPW * MPAD, BPW * MPAD)], msk_v)
        zeros = jnp.zeros((L,), jnp.float32)

        def store_slot(buf, sl, sem):
            for h in range(BPS):
                pltpu.async_copy(
                    rows_v.at[buf, pl.ds(h * SP, S)],
                    out_hbm.at[wid * BPW + sl * BPS + h],
                    sem,
                )

        def wait_store_slot(buf, sl, sem):
            for h in range(BPS):
                pltpu.make_async_copy(
                    rows_v.at[buf, pl.ds(h * SP, S)],
                    out_hbm.at[wid * BPW + sl * BPS + h],
                    sem,
                ).wait()

        def body(p, carry):
            for b in range(NBUF):
                sl = p * NBUF + b
                prev = (b - 1) % NBUF

                @pl.when(jnp.logical_and(sl >= 1, sl - 1 + NBUF < SLOTS))
                def _(b=b, sl=sl, prev=prev):
                    with jax.named_scope("refill"):
                        wait_store_slot(prev, sl - 1, ssem[prev])
                        pltpu.async_copy(
                            table_hbm.at[idx_v.at[sl - 1 + NBUF]],
                            rows_v.at[prev],
                            gsem[prev],
                        )

                with jax.named_scope("gwait"):
                    pltpu.make_async_copy(
                        table_hbm.at[idx_v.at[sl]],
                        rows_v.at[b],
                        gsem[b],
                    ).wait()

                with jax.named_scope("maskz"):
                    for h in range(BPS):
                        for g in range(4):
                            nt = L if g < 3 else S - 3 * L
                            mv = msk_v[
                                pl.ds((sl * BPS + h) * MPAD + g * L, L)
                            ]
                            for t in range(nt):
                                r = h * SP + g * L + t

                                @pl.when(mv[t] != 0)
                                def _(r=r, b=b):
                                    for j in range(D // L):
                                        rows_v[b, r, pl.ds(j * L, L)] = zeros

                with jax.named_scope("sstart"):
                    store_slot(b, sl, ssem[b])

            return carry

        lax.fori_loop(0, SLOTS // NBUF, body, 0)
        for b in range(NBUF):
            wait_store_slot(b, SLOTS - NBUF + b, ssem[b])

    return k


_k = jax.jit(_build())


def kernel(node_ids, mask, emb_table):
    fill = (jnp.arange(B, dtype=jnp.int32)[:, None] * (SP - S)
            + jnp.arange(SP - S, dtype=jnp.int32)[None, :]) % (B * (SP - S))
    idx = jnp.concatenate([node_ids.astype(jnp.int32), fill], axis=1).reshape(
        NW, SLOTS, BPS * SP)
    msk = jnp.pad(mask.astype(jnp.int32), ((0, 0), (0, MPAD - S))).reshape(-1)
    return _k(emb_table, idx, msk)


# speedup vs baseline: 1.1141x; 1.0116x over previous
import functools

import jax
import jax.numpy as jnp
from jax import lax
from jax.experimental import pallas as pl
from jax.experimental.pallas import tpu as pltpu
from jax.experimental.pallas import tpu_sc as plsc

B = 1024
S = 51
SP = 56
D = 128
NW = 32
BPW = B // NW
MPAD = 64
BPS = 1
SLOTS = BPW // BPS
NBUF = 4
L = 16


def _build():
    info = plsc.get_sparse_core_info()
    nc = info.num_cores
    mesh = plsc.VectorSubcoreMesh(core_axis_name="c", subcore_axis_name="s")

    @functools.partial(
        pl.kernel,
        mesh=mesh,
        out_type=jax.ShapeDtypeStruct((B, S, D), jnp.float32),
        scratch_types=[
            pltpu.VMEM((SLOTS, BPS * SP), jnp.int32),
            pltpu.VMEM((BPW * MPAD,), jnp.int32),
            pltpu.VMEM((NBUF, BPS * SP, D), jnp.float32),
        ]
        + [pltpu.SemaphoreType.DMA] * (2 * NBUF),
    )
    def k(table_hbm, idx_hbm, msk_hbm, out_hbm, idx_v, msk_v, rows_v, *sems):
        gsem = sems[:NBUF]
        ssem = sems[NBUF:]
        wid = lax.axis_index("s") * nc + lax.axis_index("c")
        pltpu.sync_copy(idx_hbm.at[wid], idx_v)
        for b in range(NBUF):
            pltpu.async_copy(table_hbm.at[idx_v.at[b]], rows_v.at[b], gsem[b])
        pltpu.sync_copy(msk_hbm.at[pl.ds(wid * BPW * MPAD, BPW * MPAD)], msk_v)
        zeros = jnp.zeros((L,), jnp.float32)

        def store_slot(buf, sl, sem):
            for h in range(BPS):
                pltpu.async_copy(
                    rows_v.at[buf, pl.ds(h * SP, S)],
                    out_hbm.at[wid * BPW + sl * BPS + h],
                    sem,
                )

        def wait_store_slot(buf, sl, sem):
            for h in range(BPS):
                pltpu.make_async_copy(
                    rows_v.at[buf, pl.ds(h * SP, S)],
                    out_hbm.at[wid * BPW + sl * BPS + h],
                    sem,
                ).wait()

        def body(p, carry):
            for b in range(NBUF):
                sl = p * NBUF + b
                prev = (b - 1) % NBUF

                prev2 = (b - 2) % NBUF

                @pl.when(jnp.logical_and(sl >= 2, sl - 2 + NBUF < SLOTS))
                def _(b=b, sl=sl, prev2=prev2):
                    with jax.named_scope("refill"):
                        wait_store_slot(prev2, sl - 2, ssem[prev2])
                        pltpu.async_copy(
                            table_hbm.at[idx_v.at[sl - 2 + NBUF]],
                            rows_v.at[prev2],
                            gsem[prev2],
                        )

                with jax.named_scope("gwait"):
                    pltpu.make_async_copy(
                        table_hbm.at[idx_v.at[sl]],
                        rows_v.at[b],
                        gsem[b],
                    ).wait()

                with jax.named_scope("maskz"):
                    for h in range(BPS):
                        for g in range(4):
                            nt = L if g < 3 else S - 3 * L
                            mv = msk_v[
                                pl.ds((sl * BPS + h) * MPAD + g * L, L)
                            ]
                            for t in range(nt):
                                r = h * SP + g * L + t

                                @pl.when(mv[t] != 0)
                                def _(r=r, b=b):
                                    for j in range(D // L):
                                        rows_v[b, r, pl.ds(j * L, L)] = zeros

                with jax.named_scope("sstart"):
                    store_slot(b, sl, ssem[b])

            return carry

        lax.fori_loop(0, SLOTS // NBUF, body, 0)
        for b in range(NBUF):
            wait_store_slot(b, SLOTS - NBUF + b, ssem[b])

    return k


_k = jax.jit(_build())


def kernel(node_ids, mask, emb_table):
    fill = (jnp.arange(B, dtype=jnp.int32)[:, None] * (SP - S)
            + jnp.arange(SP - S, dtype=jnp.int32)[None, :]) % (B * (SP - S))
    idx = jnp.concatenate([node_ids.astype(jnp.int32), fill], axis=1).reshape(
        NW, SLOTS, BPS * SP)
    msk = jnp.pad(mask.astype(jnp.int32), ((0, 0), (0, MPAD - S))).reshape(-1)
    return _k(emb_table, idx, msk)
